# Initial kernel scaffold; baseline (speedup 1.0000x reference)
import jax
import jax.numpy as jnp
from jax.experimental import pallas as pl


def kernel(x, W_in, router_weight, router_bias, codes, bias):
    raise NotImplementedError("write your pallas kernel here")


# speedup vs baseline: 2.5745x; 2.5745x over previous
import jax
import jax.numpy as jnp
from jax.experimental import pallas as pl

_D = 768
_HEADS = 12
_CELLS = 64
_HC = _HEADS * _CELLS
_CODE_SCALE = 1.0
_ROWS = 128
_DC = 8
_TBLK = 256


def _bf16_rne(v):
    return v.astype(jnp.bfloat16).astype(jnp.float32)


def _routing_kernel(win_ref, wint_ref, wft_ref, rbt_ref, codes_ref, reps_ref):
    def body(i, acc):
        lat = _bf16_rne(wint_ref[pl.ds(i * _DC, _DC), :])
        w = wft_ref[pl.ds(i * _DC, _DC), :]
        t = w[:, :, None] + lat[:, None, :]
        return jnp.maximum(acc, jnp.max(t, axis=0))

    neg = jnp.full((_HC, _ROWS), -jnp.inf, dtype=jnp.float32)
    scT = jax.lax.fori_loop(0, _D // _DC, body, neg)
    scT = scT + rbt_ref[...]

    at_parts = []
    for h in range(_HEADS):
        sc_h = jax.lax.slice(scT, (h * _CELLS, 0), ((h + 1) * _CELLS, _ROWS))
        v1 = jnp.max(sc_h, axis=0, keepdims=True)
        i1 = jnp.argmax(sc_h, axis=0, keepdims=True)
        cell_iota = jax.lax.broadcasted_iota(jnp.int32, (_CELLS, _ROWS), 0)
        is_w = cell_iota == i1
        masked = jnp.where(is_w, -jnp.inf, sc_h)
        v2 = jnp.max(masked, axis=0, keepdims=True)
        i2 = jnp.argmax(masked, axis=0, keepdims=True)
        alpha = jax.nn.sigmoid(v1 - v2)
        a_h = jnp.where(is_w, alpha, 0.0) + jnp.where(cell_iota == i2, 1.0 - alpha, 0.0)
        at_parts.append(a_h)
    at = jnp.concatenate(at_parts, axis=0)

    mixed = jax.lax.dot_general(at, codes_ref[...],
                                (((0,), (0,)), ((), ())),
                                preferred_element_type=jnp.float32)
    reps_ref[...] = _bf16_rne(win_ref[...]) + mixed * _CODE_SCALE


def _mm_kernel(x_ref, reps_ref, bias_ref, out_ref):
    reps = reps_ref[...]
    hidden = jnp.dot(x_ref[...], reps, preferred_element_type=jnp.float32)
    out = jax.lax.dot_general(hidden, reps, (((1,), (1,)), ((), ())),
                              preferred_element_type=jnp.float32)
    out_ref[...] = jnp.maximum(out + bias_ref[...], 0.0)


def kernel(x, W_in, router_weight, router_bias, codes, bias):
    n_features, d = W_in.shape
    heads, cells, _ = router_weight.shape
    hc = heads * cells
    wft = router_weight.reshape(hc, d).T
    rbt = router_bias.reshape(hc, 1)
    codes_flat = codes.reshape(hc, d)
    wint = W_in.T

    reps = pl.pallas_call(
        _routing_kernel,
        grid=(n_features // _ROWS,),
        in_specs=[
            pl.BlockSpec((_ROWS, d), lambda i: (i, 0)),
            pl.BlockSpec((d, _ROWS), lambda i: (0, i)),
            pl.BlockSpec((d, hc), lambda i: (0, 0)),
            pl.BlockSpec((hc, 1), lambda i: (0, 0)),
            pl.BlockSpec((hc, d), lambda i: (0, 0)),
        ],
        out_specs=pl.BlockSpec((_ROWS, d), lambda i: (i, 0)),
        out_shape=jax.ShapeDtypeStruct((n_features, d), jnp.float32),
    )(W_in, wint, wft, rbt, codes_flat)

    tokens = x.shape[0]
    bias2d = bias.reshape(1, n_features)
    out = pl.pallas_call(
        _mm_kernel,
        grid=(tokens // _TBLK,),
        in_specs=[
            pl.BlockSpec((_TBLK, n_features), lambda i: (i, 0)),
            pl.BlockSpec((n_features, d), lambda i: (0, 0)),
            pl.BlockSpec((1, n_features), lambda i: (0, 0)),
        ],
        out_specs=pl.BlockSpec((_TBLK, n_features), lambda i: (i, 0)),
        out_shape=jax.ShapeDtypeStruct((tokens, n_features), jnp.float32),
    )(x, reps, bias2d)
    return out


# speedup vs baseline: 5.2868x; 2.0535x over previous
import jax
import jax.numpy as jnp
from jax.experimental import pallas as pl

_D = 768
_HEADS = 12
_CELLS = 64
_HC = _HEADS * _CELLS
_CODE_SCALE = 1.0
_ROWS = 128
_DC = 8
_TBLK = 256


def _bf16_rne(v):
    return v.astype(jnp.bfloat16).astype(jnp.float32)


def _routing_kernel(win_ref, wint_ref, wft_ref, rb_ref, codes_ref, reps_ref):
    def body(i, acc):
        latt = _bf16_rne(wint_ref[pl.ds(i * _DC, _DC), :])
        lat = jnp.transpose(latt)
        w = wft_ref[pl.ds(i * _DC, _DC), :]
        for j in range(_DC):
            acc = jnp.maximum(acc, lat[:, j:j + 1] + w[j:j + 1, :])
        return acc

    neg = jnp.full((_ROWS, _HC), -jnp.inf, dtype=jnp.float32)
    sc = jax.lax.fori_loop(0, _D // _DC, body, neg)
    sc = sc + rb_ref[...]

    a_parts = []
    for h in range(_HEADS):
        sc_h = jax.lax.slice(sc, (0, h * _CELLS), (_ROWS, (h + 1) * _CELLS))
        v1 = jnp.max(sc_h, axis=1, keepdims=True)
        i1 = jnp.argmax(sc_h, axis=1, keepdims=True)
        cell_iota = jax.lax.broadcasted_iota(jnp.int32, (_ROWS, _CELLS), 1)
        is_w = cell_iota == i1
        masked = jnp.where(is_w, -jnp.inf, sc_h)
        v2 = jnp.max(masked, axis=1, keepdims=True)
        i2 = jnp.argmax(masked, axis=1, keepdims=True)
        alpha = jax.nn.sigmoid(v1 - v2)
        a_h = jnp.where(is_w, alpha, 0.0) + jnp.where(cell_iota == i2, 1.0 - alpha, 0.0)
        a_parts.append(a_h)
    amix = jnp.concatenate(a_parts, axis=1)

    mixed = jnp.dot(amix, codes_ref[...], preferred_element_type=jnp.float32)
    reps_ref[...] = _bf16_rne(win_ref[...]) + mixed * _CODE_SCALE


def _mm_kernel(x_ref, reps_ref, bias_ref, out_ref):
    reps = reps_ref[...]
    hidden = jnp.dot(x_ref[...], reps, preferred_element_type=jnp.float32)
    out = jax.lax.dot_general(hidden, reps, (((1,), (1,)), ((), ())),
                              preferred_element_type=jnp.float32)
    out_ref[...] = jnp.maximum(out + bias_ref[...], 0.0)


def kernel(x, W_in, router_weight, router_bias, codes, bias):
    n_features, d = W_in.shape
    heads, cells, _ = router_weight.shape
    hc = heads * cells
    wft = router_weight.reshape(hc, d).T
    rb2 = router_bias.reshape(1, hc)
    codes_flat = codes.reshape(hc, d)
    wint = W_in.T

    reps = pl.pallas_call(
        _routing_kernel,
        grid=(n_features // _ROWS,),
        in_specs=[
            pl.BlockSpec((_ROWS, d), lambda i: (i, 0)),
            pl.BlockSpec((d, _ROWS), lambda i: (0, i)),
            pl.BlockSpec((d, hc), lambda i: (0, 0)),
            pl.BlockSpec((1, hc), lambda i: (0, 0)),
            pl.BlockSpec((hc, d), lambda i: (0, 0)),
        ],
        out_specs=pl.BlockSpec((_ROWS, d), lambda i: (i, 0)),
        out_shape=jax.ShapeDtypeStruct((n_features, d), jnp.float32),
    )(W_in, wint, wft, rb2, codes_flat)

    tokens = x.shape[0]
    bias2d = bias.reshape(1, n_features)
    out = pl.pallas_call(
        _mm_kernel,
        grid=(tokens // _TBLK,),
        in_specs=[
            pl.BlockSpec((_TBLK, n_features), lambda i: (i, 0)),
            pl.BlockSpec((n_features, d), lambda i: (0, 0)),
            pl.BlockSpec((1, n_features), lambda i: (0, 0)),
        ],
        out_specs=pl.BlockSpec((_TBLK, n_features), lambda i: (i, 0)),
        out_shape=jax.ShapeDtypeStruct((tokens, n_features), jnp.float32),
    )(x, reps, bias2d)
    return out


# speedup vs baseline: 6.1724x; 1.1675x over previous
import jax
import jax.numpy as jnp
from jax.experimental import pallas as pl

_D = 768
_HEADS = 12
_CELLS = 64
_HC = _HEADS * _CELLS
_CODE_SCALE = 1.0
_ROWS = 128
_DC = 16
_TBLK = 256


def _bf16_rne(v):
    return v.astype(jnp.bfloat16).astype(jnp.float32)


def _routing_kernel(win_ref, wint_ref, wft_ref, rb_ref, codes_ref, reps_ref):
    def body(i, acc):
        latt = _bf16_rne(wint_ref[pl.ds(i * _DC, _DC), :])
        lat = jnp.transpose(latt)
        w = wft_ref[pl.ds(i * _DC, _DC), :]
        for j in range(_DC):
            acc = jnp.maximum(acc, lat[:, j:j + 1] + w[j:j + 1, :])
        return acc

    neg = jnp.full((_ROWS, _HC), -jnp.inf, dtype=jnp.float32)
    sc = jax.lax.fori_loop(0, _D // _DC, body, neg)
    sc = sc + rb_ref[...]

    a_parts = []
    for h in range(_HEADS):
        sc_h = jax.lax.slice(sc, (0, h * _CELLS), (_ROWS, (h + 1) * _CELLS))
        v1 = jnp.max(sc_h, axis=1, keepdims=True)
        i1 = jnp.argmax(sc_h, axis=1, keepdims=True)
        cell_iota = jax.lax.broadcasted_iota(jnp.int32, (_ROWS, _CELLS), 1)
        is_w = cell_iota == i1
        masked = jnp.where(is_w, -jnp.inf, sc_h)
        v2 = jnp.max(masked, axis=1, keepdims=True)
        i2 = jnp.argmax(masked, axis=1, keepdims=True)
        alpha = jax.nn.sigmoid(v1 - v2)
        a_h = jnp.where(is_w, alpha, 0.0) + jnp.where(cell_iota == i2, 1.0 - alpha, 0.0)
        a_parts.append(a_h)
    amix = jnp.concatenate(a_parts, axis=1)

    mixed = jnp.dot(amix, codes_ref[...], preferred_element_type=jnp.float32)
    reps_ref[...] = _bf16_rne(win_ref[...]) + mixed * _CODE_SCALE


def _mm_kernel(x_ref, reps_ref, bias_ref, out_ref):
    reps = reps_ref[...]
    hidden = jnp.dot(x_ref[...], reps, preferred_element_type=jnp.float32)
    out = jax.lax.dot_general(hidden, reps, (((1,), (1,)), ((), ())),
                              preferred_element_type=jnp.float32)
    out_ref[...] = jnp.maximum(out + bias_ref[...], 0.0)


def kernel(x, W_in, router_weight, router_bias, codes, bias):
    n_features, d = W_in.shape
    heads, cells, _ = router_weight.shape
    hc = heads * cells
    wft = router_weight.reshape(hc, d).T
    rb2 = router_bias.reshape(1, hc)
    codes_flat = codes.reshape(hc, d)
    wint = W_in.T

    reps = pl.pallas_call(
        _routing_kernel,
        grid=(n_features // _ROWS,),
        in_specs=[
            pl.BlockSpec((_ROWS, d), lambda i: (i, 0)),
            pl.BlockSpec((d, _ROWS), lambda i: (0, i)),
            pl.BlockSpec((d, hc), lambda i: (0, 0)),
            pl.BlockSpec((1, hc), lambda i: (0, 0)),
            pl.BlockSpec((hc, d), lambda i: (0, 0)),
        ],
        out_specs=pl.BlockSpec((_ROWS, d), lambda i: (i, 0)),
        out_shape=jax.ShapeDtypeStruct((n_features, d), jnp.float32),
    )(W_in, wint, wft, rb2, codes_flat)

    tokens = x.shape[0]
    bias2d = bias.reshape(1, n_features)
    out = pl.pallas_call(
        _mm_kernel,
        grid=(tokens // _TBLK,),
        in_specs=[
            pl.BlockSpec((_TBLK, n_features), lambda i: (i, 0)),
            pl.BlockSpec((n_features, d), lambda i: (0, 0)),
            pl.BlockSpec((1, n_features), lambda i: (0, 0)),
        ],
        out_specs=pl.BlockSpec((_TBLK, n_features), lambda i: (i, 0)),
        out_shape=jax.ShapeDtypeStruct((tokens, n_features), jnp.float32),
    )(x, reps, bias2d)
    return out


# speedup vs baseline: 7.2310x; 1.1715x over previous
import jax
import jax.numpy as jnp
from jax.experimental import pallas as pl

_D = 768
_HEADS = 12
_CELLS = 64
_HC = _HEADS * _CELLS
_CODE_SCALE = 1.0
_ROWS = 128
_DC = 32
_TBLK = 256


def _bf16_rne(v):
    return v.astype(jnp.bfloat16).astype(jnp.float32)


def _routing_kernel(win_ref, wint_ref, wft_ref, rb_ref, codes_ref, reps_ref):
    def body(i, acc):
        latt = _bf16_rne(wint_ref[pl.ds(i * _DC, _DC), :])
        lat = jnp.transpose(latt)
        w = wft_ref[pl.ds(i * _DC, _DC), :]
        for j in range(_DC):
            acc = jnp.maximum(acc, lat[:, j:j + 1] + w[j:j + 1, :])
        return acc

    neg = jnp.full((_ROWS, _HC), -jnp.inf, dtype=jnp.float32)
    sc = jax.lax.fori_loop(0, _D // _DC, body, neg)
    sc = sc + rb_ref[...]

    a_parts = []
    for h in range(_HEADS):
        sc_h = jax.lax.slice(sc, (0, h * _CELLS), (_ROWS, (h + 1) * _CELLS))
        v1 = jnp.max(sc_h, axis=1, keepdims=True)
        i1 = jnp.argmax(sc_h, axis=1, keepdims=True)
        cell_iota = jax.lax.broadcasted_iota(jnp.int32, (_ROWS, _CELLS), 1)
        is_w = cell_iota == i1
        masked = jnp.where(is_w, -jnp.inf, sc_h)
        v2 = jnp.max(masked, axis=1, keepdims=True)
        i2 = jnp.argmax(masked, axis=1, keepdims=True)
        alpha = jax.nn.sigmoid(v1 - v2)
        a_h = jnp.where(is_w, alpha, 0.0) + jnp.where(cell_iota == i2, 1.0 - alpha, 0.0)
        a_parts.append(a_h)
    amix = jnp.concatenate(a_parts, axis=1)

    mixed = jnp.dot(amix, codes_ref[...], preferred_element_type=jnp.float32)
    reps_ref[...] = _bf16_rne(win_ref[...]) + mixed * _CODE_SCALE


def _mm_kernel(x_ref, reps_ref, bias_ref, out_ref):
    reps = reps_ref[...]
    hidden = jnp.dot(x_ref[...], reps, preferred_element_type=jnp.float32)
    out = jax.lax.dot_general(hidden, reps, (((1,), (1,)), ((), ())),
                              preferred_element_type=jnp.float32)
    out_ref[...] = jnp.maximum(out + bias_ref[...], 0.0)


def kernel(x, W_in, router_weight, router_bias, codes, bias):
    n_features, d = W_in.shape
    heads, cells, _ = router_weight.shape
    hc = heads * cells
    wft = router_weight.reshape(hc, d).T
    rb2 = router_bias.reshape(1, hc)
    codes_flat = codes.reshape(hc, d)
    wint = W_in.T

    reps = pl.pallas_call(
        _routing_kernel,
        grid=(n_features // _ROWS,),
        in_specs=[
            pl.BlockSpec((_ROWS, d), lambda i: (i, 0)),
            pl.BlockSpec((d, _ROWS), lambda i: (0, i)),
            pl.BlockSpec((d, hc), lambda i: (0, 0)),
            pl.BlockSpec((1, hc), lambda i: (0, 0)),
            pl.BlockSpec((hc, d), lambda i: (0, 0)),
        ],
        out_specs=pl.BlockSpec((_ROWS, d), lambda i: (i, 0)),
        out_shape=jax.ShapeDtypeStruct((n_features, d), jnp.float32),
    )(W_in, wint, wft, rb2, codes_flat)

    tokens = x.shape[0]
    bias2d = bias.reshape(1, n_features)
    out = pl.pallas_call(
        _mm_kernel,
        grid=(tokens // _TBLK,),
        in_specs=[
            pl.BlockSpec((_TBLK, n_features), lambda i: (i, 0)),
            pl.BlockSpec((n_features, d), lambda i: (0, 0)),
            pl.BlockSpec((1, n_features), lambda i: (0, 0)),
        ],
        out_specs=pl.BlockSpec((_TBLK, n_features), lambda i: (i, 0)),
        out_shape=jax.ShapeDtypeStruct((tokens, n_features), jnp.float32),
    )(x, reps, bias2d)
    return out


# speedup vs baseline: 7.5588x; 1.0453x over previous
import jax
import jax.numpy as jnp
from jax.experimental import pallas as pl

_D = 768
_HEADS = 12
_CELLS = 64
_HC = _HEADS * _CELLS
_CODE_SCALE = 1.0
_ROWS = 128
_DC = 64
_TBLK = 256


def _bf16_rne(v):
    return v.astype(jnp.bfloat16).astype(jnp.float32)


def _routing_kernel(win_ref, wint_ref, wft_ref, rb_ref, codes_ref, reps_ref):
    def body(i, acc):
        latt = _bf16_rne(wint_ref[pl.ds(i * _DC, _DC), :])
        lat = jnp.transpose(latt)
        w = wft_ref[pl.ds(i * _DC, _DC), :]
        for j in range(_DC):
            acc = jnp.maximum(acc, lat[:, j:j + 1] + w[j:j + 1, :])
        return acc

    neg = jnp.full((_ROWS, _HC), -jnp.inf, dtype=jnp.float32)
    sc = jax.lax.fori_loop(0, _D // _DC, body, neg)
    sc = sc + rb_ref[...]

    a_parts = []
    for h in range(_HEADS):
        sc_h = jax.lax.slice(sc, (0, h * _CELLS), (_ROWS, (h + 1) * _CELLS))
        v1 = jnp.max(sc_h, axis=1, keepdims=True)
        i1 = jnp.argmax(sc_h, axis=1, keepdims=True)
        cell_iota = jax.lax.broadcasted_iota(jnp.int32, (_ROWS, _CELLS), 1)
        is_w = cell_iota == i1
        masked = jnp.where(is_w, -jnp.inf, sc_h)
        v2 = jnp.max(masked, axis=1, keepdims=True)
        i2 = jnp.argmax(masked, axis=1, keepdims=True)
        alpha = jax.nn.sigmoid(v1 - v2)
        a_h = jnp.where(is_w, alpha, 0.0) + jnp.where(cell_iota == i2, 1.0 - alpha, 0.0)
        a_parts.append(a_h)
    amix = jnp.concatenate(a_parts, axis=1)

    mixed = jnp.dot(amix, codes_ref[...], preferred_element_type=jnp.float32)
    reps_ref[...] = _bf16_rne(win_ref[...]) + mixed * _CODE_SCALE


def _mm_kernel(x_ref, reps_ref, bias_ref, out_ref):
    reps = reps_ref[...]
    hidden = jnp.dot(x_ref[...], reps, preferred_element_type=jnp.float32)
    out = jax.lax.dot_general(hidden, reps, (((1,), (1,)), ((), ())),
                              preferred_element_type=jnp.float32)
    out_ref[...] = jnp.maximum(out + bias_ref[...], 0.0)


def kernel(x, W_in, router_weight, router_bias, codes, bias):
    n_features, d = W_in.shape
    heads, cells, _ = router_weight.shape
    hc = heads * cells
    wft = router_weight.reshape(hc, d).T
    rb2 = router_bias.reshape(1, hc)
    codes_flat = codes.reshape(hc, d)
    wint = W_in.T

    reps = pl.pallas_call(
        _routing_kernel,
        grid=(n_features // _ROWS,),
        in_specs=[
            pl.BlockSpec((_ROWS, d), lambda i: (i, 0)),
            pl.BlockSpec((d, _ROWS), lambda i: (0, i)),
            pl.BlockSpec((d, hc), lambda i: (0, 0)),
            pl.BlockSpec((1, hc), lambda i: (0, 0)),
            pl.BlockSpec((hc, d), lambda i: (0, 0)),
        ],
        out_specs=pl.BlockSpec((_ROWS, d), lambda i: (i, 0)),
        out_shape=jax.ShapeDtypeStruct((n_features, d), jnp.float32),
    )(W_in, wint, wft, rb2, codes_flat)

    tokens = x.shape[0]
    bias2d = bias.reshape(1, n_features)
    out = pl.pallas_call(
        _mm_kernel,
        grid=(tokens // _TBLK,),
        in_specs=[
            pl.BlockSpec((_TBLK, n_features), lambda i: (i, 0)),
            pl.BlockSpec((n_features, d), lambda i: (0, 0)),
            pl.BlockSpec((1, n_features), lambda i: (0, 0)),
        ],
        out_specs=pl.BlockSpec((_TBLK, n_features), lambda i: (i, 0)),
        out_shape=jax.ShapeDtypeStruct((tokens, n_features), jnp.float32),
    )(x, reps, bias2d)
    return out
